# baseline (speedup 1.0000x reference)
import functools

import jax
import jax.numpy as jnp
from jax import lax
from jax.experimental import pallas as pl
from jax.experimental.pallas import tpu as pltpu
from jax.experimental.pallas import tpu_sc as plsc

_N = 10000
_E = 320000
_D = 128

_NC = 2
_NS = 16
_NW = _NC * _NS
_PER_W = _E // _NW
_CH = 400
_DP = 64
_CHA = 256
_CHB = 144
_N_PAD = 10240
_N_HALF = _N_PAD // _NC
_ROWS_T = _N_HALF // _NS
_EH0 = 153600
_EH1 = _E - _EH0


def _sc_gather_count(x, row, col, eh, off):
    mesh = plsc.VectorSubcoreMesh(core_axis_name="c", subcore_axis_name="s")
    per_w = eh // _NW
    GA, GB = 208, 192
    NPAIR = per_w // (GA + GB)
    gbufs_ch = (GA, GB)

    @functools.partial(
        pl.kernel,
        out_type=(
            jax.ShapeDtypeStruct((eh, _D), jnp.float32),
            jax.ShapeDtypeStruct((_NW * _N_PAD,), jnp.float32),
        ),
        mesh=mesh,
        scratch_types=[
            pltpu.VMEM((per_w,), jnp.int32),
            pltpu.VMEM((per_w,), jnp.int32),
            pltpu.VMEM((GA, _D), jnp.float32),
            pltpu.VMEM((GB, _D), jnp.float32),
            pltpu.VMEM((_N_PAD,), jnp.float32),
            pltpu.SemaphoreType.DMA,
            pltpu.SemaphoreType.DMA,
            pltpu.SemaphoreType.DMA,
            pltpu.SemaphoreType.DMA,
        ],
        compiler_params=pltpu.CompilerParams(needs_layout_passes=False),
    )
    def k(x_hbm, row_hbm, col_hbm, out_hbm, cnt_hbm, idx_v, col_v, rows_a,
          rows_b, hist_v, sem_i, sem_g, sem_wa, sem_wb):
        wid = lax.axis_index("s") * _NC + lax.axis_index("c")
        span = wid * per_w
        zeros16 = jnp.zeros((16,), jnp.float32)
        ones16 = jnp.ones((16,), jnp.float32)

        cp_i = pltpu.async_copy(
            row_hbm.at[pl.ds(off + span, per_w)], idx_v, sem_i)
        cp_c = pltpu.async_copy(
            col_hbm.at[pl.ds(off + span, per_w)], col_v, sem_i)

        def zbody(i, carry):
            hist_v[pl.ds(i * 16, 16)] = zeros16
            return carry

        lax.fori_loop(0, _N_PAD // 16, zbody, 0)
        cp_i.wait()
        cp_c.wait()

        bufs = ((rows_a, sem_wa, GA, 0), (rows_b, sem_wb, GB, GA))

        def body(p, carry):
            for rows_v, sem_w, ch, coff in bufs:
                base = span + p * (GA + GB) + coff

                @pl.when(p > 0)
                def _():
                    pltpu.make_async_copy(
                        rows_v, out_hbm.at[pl.ds(base - GA - GB, ch)],
                        sem_w).wait()

                pltpu.async_copy(
                    x_hbm.at[idx_v.at[pl.ds(base - span, ch)]], rows_v,
                    sem_g).wait()
                pltpu.async_copy(rows_v, out_hbm.at[pl.ds(base, ch)], sem_w)
            return carry

        lax.fori_loop(0, NPAIR, body, 0)
        for rows_v, sem_w, ch, coff in bufs:
            base = span + (NPAIR - 1) * (GA + GB) + coff
            pltpu.make_async_copy(
                rows_v, out_hbm.at[pl.ds(base, ch)], sem_w).wait()

        def hbody(j, c2):
            idx16 = col_v[pl.ds(j * 16, 16)]
            plsc.addupdate_scatter(hist_v, [idx16], ones16)
            return c2

        lax.fori_loop(0, per_w // 16, hbody, 0)
        pltpu.sync_copy(hist_v, cnt_hbm.at[pl.ds(wid * _N_PAD, _N_PAD)])

    return k(x, row, col)


def _sc_scatter(h0, h1, col, zeros_tile):
    mesh = plsc.VectorSubcoreMesh(core_axis_name="c", subcore_axis_name="s")

    @functools.partial(
        pl.kernel,
        out_type=jax.ShapeDtypeStruct((_NC, _N_HALF, _D), jnp.float32),
        mesh=mesh,
        scratch_types=[
            pltpu.VMEM((_CHA,), jnp.int32),
            pltpu.VMEM((_CHB,), jnp.int32),
            pltpu.VMEM((_CHA, _D), jnp.float32),
            pltpu.VMEM((_CHB, _D), jnp.float32),
            pltpu.VMEM_SHARED((_N_HALF + 8, _D), jnp.float32),
            pltpu.SemaphoreType.DMA,
            pltpu.SemaphoreType.DMA,
            pltpu.SemaphoreType.DMA,
            pltpu.SemaphoreType.DMA,
            pltpu.SemaphoreType.DMA,
            pltpu.SemaphoreType.DMA,
        ],
        compiler_params=pltpu.CompilerParams(needs_layout_passes=False),
    )
    def k(h0_hbm, h1_hbm, col_hbm, zero_hbm, out_hbm, idx_a, idx_b, rows_a,
          rows_b, acc_sh, sem_ia, sem_ib, sem_ra, sem_rb, sem_sa, sem_sb):
        c = lax.axis_index("c")
        s = lax.axis_index("s")
        nbase = c * _N_HALF
        bufs = ((idx_a, rows_a, _CHA, 0, sem_ia, sem_ra, sem_sa),
                (idx_b, rows_b, _CHB, _CHA, sem_ib, sem_rb, sem_sb))

        def wait_stream(bset):
            idx_v, rows_v = bset[0], bset[1]
            pltpu.make_async_copy(rows_v, acc_sh.at[idx_v], bset[6]).wait()

        def run_half(h_hbm, col_off, per_t):
            npair = per_t // (_CHA + _CHB)

            def start_dma(pair, bset):
                idx_v, rows_v, ch, off, sem_i, sem_r, _ = bset
                base = s * per_t + pair * (_CHA + _CHB) + off
                pltpu.async_copy(
                    col_hbm.at[pl.ds(col_off + base, ch)], idx_v, sem_i)
                pltpu.async_copy(h_hbm.at[pl.ds(base, ch)], rows_v, sem_r)

            def wait_dma(pair, bset):
                idx_v, rows_v, ch, off, sem_i, sem_r, _ = bset
                base = s * per_t + pair * (_CHA + _CHB) + off
                pltpu.make_async_copy(
                    col_hbm.at[pl.ds(col_off + base, ch)], idx_v,
                    sem_i).wait()
                pltpu.make_async_copy(
                    h_hbm.at[pl.ds(base, ch)], rows_v, sem_r).wait()

            start_dma(0, bufs[0])

            def body(io, carry):
                for b in range(2):
                    bset = bufs[b]
                    idx_v, rows_v, ch = bset[0], bset[1], bset[2]
                    sem_s = bset[6]
                    wait_dma(io, bset)

                    def tbody(j, c2):
                        c16 = idx_v[pl.ds(j * 16, 16)]
                        loc = c16 - nbase
                        ok = jnp.logical_and(loc >= 0, loc < _N_HALF)
                        idx_v[pl.ds(j * 16, 16)] = jnp.where(ok, loc, _N_HALF)
                        return c2

                    lax.fori_loop(0, ch // 16, tbody, 0)
                    pltpu.async_copy(rows_v, acc_sh.at[idx_v], sem_s,
                                     add=True)
                    other = bufs[1 - b]
                    if b == 0:
                        @pl.when(io >= 1)
                        def _():
                            wait_stream(other)

                        start_dma(io, other)
                    else:
                        @pl.when(io < npair - 1)
                        def _():
                            wait_stream(other)
                            start_dma(io + 1, other)

                return carry

            lax.fori_loop(0, npair, body, 0)
            wait_stream(bufs[0])
            wait_stream(bufs[1])

        pltpu.sync_copy(zero_hbm, acc_sh.at[pl.ds(s * _ROWS_T, _ROWS_T)])

        @pl.when(s == 0)
        def _():
            pltpu.sync_copy(zero_hbm.at[pl.ds(0, 8)],
                            acc_sh.at[pl.ds(_N_HALF, 8)])

        plsc.subcore_barrier()
        run_half(h0_hbm, 0, _EH0 // _NS)
        run_half(h1_hbm, _EH0, _EH1 // _NS)
        plsc.subcore_barrier()
        pltpu.sync_copy(
            acc_sh.at[pl.ds(s * _ROWS_T, _ROWS_T)],
            out_hbm.at[c].at[pl.ds(s * _ROWS_T, _ROWS_T)],
        )

    return k(h0, h1, col, zeros_tile)


def _edge_mlp(g, ea, w0x, w0e, b0, w1, b1, w2, b2, w3, b3, w4):
    eh = g.shape[0]
    BE = eh // 80
    grid = (eh // BE,)

    dot = functools.partial(jnp.dot, preferred_element_type=jnp.float32)
    bf = jnp.bfloat16

    def body(g_ref, e_ref, w0x_r, w0e_r, b0_r, w1_r, b1_r, w2_r, b2_r,
             w3_r, b3_r, w4_r, out_ref):
        g = g_ref[...]
        h = dot(g, w0x_r[...]) + dot(e_ref[...], w0e_r[...]) + b0_r[...]
        h = jnp.maximum(h, 0.0).astype(bf)
        h = jnp.maximum(dot(h, w1_r[...]) + b1_r[...], 0.0).astype(bf)
        h = jnp.maximum(dot(h, w2_r[...]) + b2_r[...], 0.0).astype(bf)
        h = jnp.maximum(dot(h, w3_r[...]) + b3_r[...], 0.0).astype(bf)
        out_ref[...] = dot(h, w4_r[...])

    full = lambda shape: pl.BlockSpec(shape, lambda i: (0,) * len(shape))
    return pl.pallas_call(
        body,
        grid=grid,
        in_specs=[
            pl.BlockSpec((BE, _D), lambda i: (i, 0)),
            pl.BlockSpec((BE, 16), lambda i: (i, 0)),
            full((_D, 128)), full((16, 128)), full((1, 128)),
            full((128, 128)), full((1, 128)),
            full((128, 128)), full((1, 128)),
            full((128, 144)), full((1, 144)),
            full((144, 128)),
        ],
        out_specs=pl.BlockSpec((BE, _D), lambda i: (i, 0)),
        out_shape=jax.ShapeDtypeStruct((eh, _D), jnp.float32),
    )(g, ea, w0x, w0e, b0, w1, b1, w2, b2, w3, b3, w4)


def _node_mlp(x, partials, cnt_parts, ga, w0x, w0g, b0, w1, b1, w2, b2,
              w3, b3):
    BN = 2048
    grid = (_N_PAD // BN,)

    def body(x_ref, p_ref, c_ref, ones_r, ga_r, w0x_r, w0g_r, b0_r, w1_r,
             b1_r, w2_r, b2_r, w3_r, b3_r, out_ref):
        s = p_ref[...]
        cnt = lax.dot_general(c_ref[...], ones_r[...],
                              (((0,), (0,)), ((), ())))
        agg = s * (1.0 / jnp.maximum(cnt, 1.0))
        gb = ga_r[...] @ w0g_r[...] + b0_r[...]
        h = jnp.maximum(x_ref[...] @ w0x_r[...] + agg + gb, 0.0)
        h = jnp.maximum(h @ w1_r[...] + b1_r[...], 0.0)
        h = jnp.maximum(h @ w2_r[...] + b2_r[...], 0.0)
        out_ref[...] = h @ w3_r[...] + b3_r[...]

    full = lambda shape: pl.BlockSpec(shape, lambda i: (0,) * len(shape))
    return pl.pallas_call(
        body,
        grid=grid,
        in_specs=[
            pl.BlockSpec((BN, _D), lambda i: (i, 0)),
            pl.BlockSpec((BN, _D), lambda i: (i, 0)),
            pl.BlockSpec((2 * _NW, BN), lambda i: (0, i)),
            full((2 * _NW, 1)),
            full((1, 16)),
            full((_D, 128)), full((16, 128)), full((1, 128)),
            full((128, 128)), full((1, 128)),
            full((128, 128)), full((1, 128)),
            full((128, 128)), full((1, 128)),
        ],
        out_specs=pl.BlockSpec((BN, 128), lambda i: (i, 0)),
        out_shape=jax.ShapeDtypeStruct((_N_PAD, 128), jnp.float32),
    )(x, partials, cnt_parts, jnp.ones((2 * _NW, 1), jnp.float32), ga, w0x,
      w0g, b0, w1, b1, w2, b2, w3, b3)


def kernel(x, edge_index, edge_attr, global_attr,
           W10, b10, W11, b11, W12, b12, W13, b13,
           W20, b20, W21, b21, W22, b22, W23, b23):
    row = edge_index[0]
    col = edge_index[1]

    w10x = W10[:_D]
    w10e = W10[_D:]
    w20x = W20[:_D]
    w20a = W20[_D:_D + 144]
    w20g = W20[_D + 144:]

    bf = jnp.bfloat16
    ea = edge_attr.astype(bf)
    ew = (w10x.astype(bf), w10e.astype(bf), b10.reshape(1, -1),
          W11.astype(bf), b11.reshape(1, -1),
          W12.astype(bf), b12.reshape(1, -1),
          W13.astype(bf), b13.reshape(1, -1),
          w20a.astype(bf))
    x_packed = jnp.concatenate([
        lax.bitcast_convert_type(x.astype(bf).reshape(_N, _DP, 2),
                                 jnp.float32),
        jnp.zeros((_N, _D - _DP), jnp.float32)], axis=1)
    g0, cnt0 = _sc_gather_count(x_packed, row, col, _EH0, 0)
    g1, cnt1 = _sc_gather_count(x_packed, row, col, _EH1, _EH0)
    gb0 = lax.bitcast_convert_type(g0, bf).reshape(_EH0, 2 * _D)
    gb1 = lax.bitcast_convert_type(g1, bf).reshape(_EH1, 2 * _D)
    h0 = _edge_mlp(gb0, ea[:_EH0], *ew)
    h1 = _edge_mlp(gb1, ea[_EH0:], *ew)
    cnt_parts = jnp.concatenate([cnt0, cnt1]).reshape(2 * _NW, _N_PAD)
    zeros_tile = jnp.zeros((_ROWS_T, _D), jnp.float32)
    sums = _sc_scatter(h0, h1, col, zeros_tile).reshape(_N_PAD, _D)
    x_pad = jnp.concatenate(
        [x, jnp.zeros((_N_PAD - _N, _D), jnp.float32)], axis=0)
    out = _node_mlp(x_pad, sums, cnt_parts, global_attr,
                    w20x, w20g, b20.reshape(1, -1),
                    W21, b21.reshape(1, -1),
                    W22, b22.reshape(1, -1),
                    W23, b23.reshape(1, -1))
    return out[:_N]


# speedup vs baseline: 2.7945x; 2.7945x over previous
import functools

import jax
import jax.numpy as jnp
from jax import lax
from jax.experimental import pallas as pl
from jax.experimental.pallas import tpu as pltpu
from jax.experimental.pallas import tpu_sc as plsc

_N = 10000
_E = 320000
_D = 128

_NC = 2
_NS = 16
_NW = _NC * _NS
_PER_W = _E // _NW
_CH = 400
_DP = 64
_CHA = 256
_CHB = 144
_N_PAD = 10240
_N_HALF = _N_PAD // _NC
_ROWS_T = _N_HALF // _NS
_EH0 = 153600
_EH1 = _E - _EH0


def _sc_gather_count(x, row, col, eh, off):
    mesh = plsc.VectorSubcoreMesh(core_axis_name="c", subcore_axis_name="s")
    per_w = eh // _NW
    GA, GB = 208, 192
    NPAIR = per_w // (GA + GB)
    gbufs_ch = (GA, GB)

    @functools.partial(
        pl.kernel,
        out_type=(
            jax.ShapeDtypeStruct((eh, _D), jnp.float32),
            jax.ShapeDtypeStruct((_NW * _N_PAD,), jnp.float32),
        ),
        mesh=mesh,
        scratch_types=[
            pltpu.VMEM((per_w,), jnp.int32),
            pltpu.VMEM((per_w,), jnp.int32),
            pltpu.VMEM((GA, _D), jnp.float32),
            pltpu.VMEM((GB, _D), jnp.float32),
            pltpu.VMEM((_N_PAD,), jnp.float32),
            pltpu.SemaphoreType.DMA,
            pltpu.SemaphoreType.DMA,
            pltpu.SemaphoreType.DMA,
            pltpu.SemaphoreType.DMA,
        ],
        compiler_params=pltpu.CompilerParams(needs_layout_passes=False),
    )
    def k(x_hbm, row_hbm, col_hbm, out_hbm, cnt_hbm, idx_v, col_v, rows_a,
          rows_b, hist_v, sem_i, sem_g, sem_wa, sem_wb):
        wid = lax.axis_index("s") * _NC + lax.axis_index("c")
        span = wid * per_w
        zeros16 = jnp.zeros((16,), jnp.float32)
        ones16 = jnp.ones((16,), jnp.float32)

        cp_i = pltpu.async_copy(
            row_hbm.at[pl.ds(off + span, per_w)], idx_v, sem_i)
        cp_c = pltpu.async_copy(
            col_hbm.at[pl.ds(off + span, per_w)], col_v, sem_i)

        def zbody(i, carry):
            hist_v[pl.ds(i * 16, 16)] = zeros16
            return carry

        lax.fori_loop(0, _N_PAD // 16, zbody, 0)
        cp_i.wait()
        cp_c.wait()

        bufs = ((rows_a, sem_wa, GA, 0), (rows_b, sem_wb, GB, GA))

        def body(p, carry):
            for rows_v, sem_w, ch, coff in bufs:
                base = span + p * (GA + GB) + coff

                @pl.when(p > 0)
                def _():
                    pltpu.make_async_copy(
                        rows_v, out_hbm.at[pl.ds(base - GA - GB, ch)],
                        sem_w).wait()

                pltpu.async_copy(
                    x_hbm.at[idx_v.at[pl.ds(base - span, ch)]], rows_v,
                    sem_g).wait()
                pltpu.async_copy(rows_v, out_hbm.at[pl.ds(base, ch)], sem_w)
            return carry

        lax.fori_loop(0, NPAIR, body, 0)
        for rows_v, sem_w, ch, coff in bufs:
            base = span + (NPAIR - 1) * (GA + GB) + coff
            pltpu.make_async_copy(
                rows_v, out_hbm.at[pl.ds(base, ch)], sem_w).wait()

        def hbody(j, c2):
            idx16 = col_v[pl.ds(j * 16, 16)]
            plsc.addupdate_scatter(hist_v, [idx16], ones16)
            return c2

        lax.fori_loop(0, per_w // 16, hbody, 0)
        pltpu.sync_copy(hist_v, cnt_hbm.at[pl.ds(wid * _N_PAD, _N_PAD)])

    return k(x, row, col)


def _sc_scatter(h0, h1, col, zeros_tile):
    mesh = plsc.VectorSubcoreMesh(core_axis_name="c", subcore_axis_name="s")

    @functools.partial(
        pl.kernel,
        out_type=jax.ShapeDtypeStruct((_NC, _N_HALF, _D), jnp.float32),
        mesh=mesh,
        scratch_types=[
            pltpu.VMEM((_CHA,), jnp.int32),
            pltpu.VMEM((_CHB,), jnp.int32),
            pltpu.VMEM((_CHA, _D), jnp.float32),
            pltpu.VMEM((_CHB, _D), jnp.float32),
            pltpu.VMEM_SHARED((_N_HALF + 8, _D), jnp.float32),
            pltpu.SemaphoreType.DMA,
            pltpu.SemaphoreType.DMA,
            pltpu.SemaphoreType.DMA,
            pltpu.SemaphoreType.DMA,
            pltpu.SemaphoreType.DMA,
            pltpu.SemaphoreType.DMA,
        ],
        compiler_params=pltpu.CompilerParams(needs_layout_passes=False),
    )
    def k(h0_hbm, h1_hbm, col_hbm, zero_hbm, out_hbm, idx_a, idx_b, rows_a,
          rows_b, acc_sh, sem_ia, sem_ib, sem_ra, sem_rb, sem_sa, sem_sb):
        c = lax.axis_index("c")
        s = lax.axis_index("s")
        nbase = c * _N_HALF
        bufs = ((idx_a, rows_a, _CHA, 0, sem_ia, sem_ra, sem_sa),
                (idx_b, rows_b, _CHB, _CHA, sem_ib, sem_rb, sem_sb))

        def wait_stream(bset):
            idx_v, rows_v = bset[0], bset[1]
            pltpu.make_async_copy(rows_v, acc_sh.at[idx_v], bset[6]).wait()

        def run_half(h_hbm, col_off, per_t):
            npair = per_t // (_CHA + _CHB)

            def start_dma(pair, bset):
                idx_v, rows_v, ch, off, sem_i, sem_r, _ = bset
                base = s * per_t + pair * (_CHA + _CHB) + off
                pltpu.async_copy(
                    col_hbm.at[pl.ds(col_off + base, ch)], idx_v, sem_i)
                pltpu.async_copy(h_hbm.at[pl.ds(base, ch)], rows_v, sem_r)

            def wait_dma(pair, bset):
                idx_v, rows_v, ch, off, sem_i, sem_r, _ = bset
                base = s * per_t + pair * (_CHA + _CHB) + off
                pltpu.make_async_copy(
                    col_hbm.at[pl.ds(col_off + base, ch)], idx_v,
                    sem_i).wait()
                pltpu.make_async_copy(
                    h_hbm.at[pl.ds(base, ch)], rows_v, sem_r).wait()

            start_dma(0, bufs[0])

            def body(io, carry):
                for b in range(2):
                    bset = bufs[b]
                    idx_v, rows_v, ch = bset[0], bset[1], bset[2]
                    sem_s = bset[6]
                    wait_dma(io, bset)

                    def tbody(j, c2):
                        c16 = idx_v[pl.ds(j * 16, 16)]
                        loc = c16 - nbase
                        ok = jnp.logical_and(loc >= 0, loc < _N_HALF)
                        idx_v[pl.ds(j * 16, 16)] = jnp.where(ok, loc, _N_HALF)
                        return c2

                    lax.fori_loop(0, ch // 16, tbody, 0)
                    pltpu.async_copy(rows_v, acc_sh.at[idx_v], sem_s,
                                     add=True)
                    other = bufs[1 - b]
                    if b == 0:
                        @pl.when(io >= 1)
                        def _():
                            wait_stream(other)

                        start_dma(io, other)
                    else:
                        @pl.when(io < npair - 1)
                        def _():
                            wait_stream(other)
                            start_dma(io + 1, other)

                return carry

            lax.fori_loop(0, npair, body, 0)
            wait_stream(bufs[0])
            wait_stream(bufs[1])

        pltpu.sync_copy(zero_hbm, acc_sh.at[pl.ds(s * _ROWS_T, _ROWS_T)])

        @pl.when(s == 0)
        def _():
            pltpu.sync_copy(zero_hbm.at[pl.ds(0, 8)],
                            acc_sh.at[pl.ds(_N_HALF, 8)])

        plsc.subcore_barrier()
        run_half(h0_hbm, 0, _EH0 // _NS)
        run_half(h1_hbm, _EH0, _EH1 // _NS)
        plsc.subcore_barrier()
        pltpu.sync_copy(
            acc_sh.at[pl.ds(s * _ROWS_T, _ROWS_T)],
            out_hbm.at[c].at[pl.ds(s * _ROWS_T, _ROWS_T)],
        )

    return k(h0, h1, col, zeros_tile)


def _edge_mlp(g, ea, w0x, w0e, b0, w1, b1, w2, b2, w3, b3, w4):
    eh = g.shape[0]
    BE = eh // 80
    grid = (eh // BE,)

    dot = functools.partial(jnp.dot, preferred_element_type=jnp.float32)
    bf = jnp.bfloat16

    def body(g_ref, e_ref, w0x_r, w0e_r, b0_r, w1_r, b1_r, w2_r, b2_r,
             w3_r, b3_r, w4_r, out_ref):
        g = g_ref[...].astype(bf)
        h = dot(g, w0x_r[...]) + dot(e_ref[...], w0e_r[...]) + b0_r[...]
        h = jnp.maximum(h, 0.0).astype(bf)
        h = jnp.maximum(dot(h, w1_r[...]) + b1_r[...], 0.0).astype(bf)
        h = jnp.maximum(dot(h, w2_r[...]) + b2_r[...], 0.0).astype(bf)
        h = jnp.maximum(dot(h, w3_r[...]) + b3_r[...], 0.0).astype(bf)
        out_ref[...] = dot(h, w4_r[...])

    full = lambda shape: pl.BlockSpec(shape, lambda i: (0,) * len(shape))
    return pl.pallas_call(
        body,
        grid=grid,
        in_specs=[
            pl.BlockSpec((BE, _D), lambda i: (i, 0)),
            pl.BlockSpec((BE, 16), lambda i: (i, 0)),
            full((_D, 128)), full((16, 128)), full((1, 128)),
            full((128, 128)), full((1, 128)),
            full((128, 128)), full((1, 128)),
            full((128, 144)), full((1, 144)),
            full((144, 128)),
        ],
        out_specs=pl.BlockSpec((BE, _D), lambda i: (i, 0)),
        out_shape=jax.ShapeDtypeStruct((eh, _D), jnp.float32),
    )(g, ea, w0x, w0e, b0, w1, b1, w2, b2, w3, b3, w4)


def _node_mlp(x, partials, cnt_parts, ga, w0x, w0g, b0, w1, b1, w2, b2,
              w3, b3):
    BN = 2048
    grid = (_N_PAD // BN,)

    def body(x_ref, p_ref, c_ref, ones_r, ga_r, w0x_r, w0g_r, b0_r, w1_r,
             b1_r, w2_r, b2_r, w3_r, b3_r, out_ref):
        s = p_ref[...]
        cnt = lax.dot_general(c_ref[...], ones_r[...],
                              (((0,), (0,)), ((), ())))
        agg = s * (1.0 / jnp.maximum(cnt, 1.0))
        gb = ga_r[...] @ w0g_r[...] + b0_r[...]
        h = jnp.maximum(x_ref[...] @ w0x_r[...] + agg + gb, 0.0)
        h = jnp.maximum(h @ w1_r[...] + b1_r[...], 0.0)
        h = jnp.maximum(h @ w2_r[...] + b2_r[...], 0.0)
        out_ref[...] = h @ w3_r[...] + b3_r[...]

    full = lambda shape: pl.BlockSpec(shape, lambda i: (0,) * len(shape))
    return pl.pallas_call(
        body,
        grid=grid,
        in_specs=[
            pl.BlockSpec((BN, _D), lambda i: (i, 0)),
            pl.BlockSpec((BN, _D), lambda i: (i, 0)),
            pl.BlockSpec((2 * _NW, BN), lambda i: (0, i)),
            full((2 * _NW, 1)),
            full((1, 16)),
            full((_D, 128)), full((16, 128)), full((1, 128)),
            full((128, 128)), full((1, 128)),
            full((128, 128)), full((1, 128)),
            full((128, 128)), full((1, 128)),
        ],
        out_specs=pl.BlockSpec((BN, 128), lambda i: (i, 0)),
        out_shape=jax.ShapeDtypeStruct((_N_PAD, 128), jnp.float32),
    )(x, partials, cnt_parts, jnp.ones((2 * _NW, 1), jnp.float32), ga, w0x,
      w0g, b0, w1, b1, w2, b2, w3, b3)


def kernel(x, edge_index, edge_attr, global_attr,
           W10, b10, W11, b11, W12, b12, W13, b13,
           W20, b20, W21, b21, W22, b22, W23, b23):
    row = edge_index[0]
    col = edge_index[1]

    w10x = W10[:_D]
    w10e = W10[_D:]
    w20x = W20[:_D]
    w20a = W20[_D:_D + 144]
    w20g = W20[_D + 144:]

    bf = jnp.bfloat16
    ea = edge_attr.astype(bf)
    ew = (w10x.astype(bf), w10e.astype(bf), b10.reshape(1, -1),
          W11.astype(bf), b11.reshape(1, -1),
          W12.astype(bf), b12.reshape(1, -1),
          W13.astype(bf), b13.reshape(1, -1),
          w20a.astype(bf))
    g0, cnt0 = _sc_gather_count(x, row, col, _EH0, 0)
    g1, cnt1 = _sc_gather_count(x, row, col, _EH1, _EH0)
    h0 = _edge_mlp(g0, ea[:_EH0], *ew)
    h1 = _edge_mlp(g1, ea[_EH0:], *ew)
    cnt_parts = jnp.concatenate([cnt0, cnt1]).reshape(2 * _NW, _N_PAD)
    zeros_tile = jnp.zeros((_ROWS_T, _D), jnp.float32)
    sums = _sc_scatter(h0, h1, col, zeros_tile).reshape(_N_PAD, _D)
    x_pad = jnp.concatenate(
        [x, jnp.zeros((_N_PAD - _N, _D), jnp.float32)], axis=0)
    out = _node_mlp(x_pad, sums, cnt_parts, global_attr,
                    w20x, w20g, b20.reshape(1, -1),
                    W21, b21.reshape(1, -1),
                    W22, b22.reshape(1, -1),
                    W23, b23.reshape(1, -1))
    return out[:_N]


# speedup vs baseline: 2.7953x; 1.0003x over previous
import functools

import jax
import jax.numpy as jnp
from jax import lax
from jax.experimental import pallas as pl
from jax.experimental.pallas import tpu as pltpu
from jax.experimental.pallas import tpu_sc as plsc

_N = 10000
_E = 320000
_D = 128

_NC = 2
_NS = 16
_NW = _NC * _NS
_PER_W = _E // _NW
_CH = 400
_DP = 64
_CHA = 256
_CHB = 144
_N_PAD = 10240
_N_HALF = _N_PAD // _NC
_ROWS_T = _N_HALF // _NS
_EH0 = 153600
_EH1 = _E - _EH0


def _sc_gather_count(x, row, col, eh, off):
    mesh = plsc.VectorSubcoreMesh(core_axis_name="c", subcore_axis_name="s")
    per_w = eh // _NW
    GA, GB = 208, 192
    NPAIR = per_w // (GA + GB)
    gbufs_ch = (GA, GB)

    @functools.partial(
        pl.kernel,
        out_type=(
            jax.ShapeDtypeStruct((eh, _D), jnp.float32),
            jax.ShapeDtypeStruct((_NW * _N_PAD,), jnp.float32),
        ),
        mesh=mesh,
        scratch_types=[
            pltpu.VMEM((per_w,), jnp.int32),
            pltpu.VMEM((per_w,), jnp.int32),
            pltpu.VMEM((GA, _D), jnp.float32),
            pltpu.VMEM((GB, _D), jnp.float32),
            pltpu.VMEM((_N_PAD,), jnp.float32),
            pltpu.SemaphoreType.DMA,
            pltpu.SemaphoreType.DMA,
            pltpu.SemaphoreType.DMA,
            pltpu.SemaphoreType.DMA,
        ],
        compiler_params=pltpu.CompilerParams(needs_layout_passes=False),
    )
    def k(x_hbm, row_hbm, col_hbm, out_hbm, cnt_hbm, idx_v, col_v, rows_a,
          rows_b, hist_v, sem_i, sem_g, sem_wa, sem_wb):
        wid = lax.axis_index("s") * _NC + lax.axis_index("c")
        span = wid * per_w
        zeros16 = jnp.zeros((16,), jnp.float32)
        ones16 = jnp.ones((16,), jnp.float32)

        cp_i = pltpu.async_copy(
            row_hbm.at[pl.ds(off + span, per_w)], idx_v, sem_i)
        cp_c = pltpu.async_copy(
            col_hbm.at[pl.ds(off + span, per_w)], col_v, sem_i)

        def zbody(i, carry):
            hist_v[pl.ds(i * 16, 16)] = zeros16
            return carry

        lax.fori_loop(0, _N_PAD // 16, zbody, 0)
        cp_i.wait()
        cp_c.wait()

        bufs = ((rows_a, sem_wa, GA, 0), (rows_b, sem_wb, GB, GA))

        def body(p, carry):
            for rows_v, sem_w, ch, coff in bufs:
                base = span + p * (GA + GB) + coff

                @pl.when(p > 0)
                def _():
                    pltpu.make_async_copy(
                        rows_v, out_hbm.at[pl.ds(base - GA - GB, ch)],
                        sem_w).wait()

                pltpu.async_copy(
                    x_hbm.at[idx_v.at[pl.ds(base - span, ch)]], rows_v,
                    sem_g).wait()
                pltpu.async_copy(rows_v, out_hbm.at[pl.ds(base, ch)], sem_w)
            return carry

        lax.fori_loop(0, NPAIR, body, 0)
        for rows_v, sem_w, ch, coff in bufs:
            base = span + (NPAIR - 1) * (GA + GB) + coff
            pltpu.make_async_copy(
                rows_v, out_hbm.at[pl.ds(base, ch)], sem_w).wait()

        def hbody(j, c2):
            idx16 = col_v[pl.ds(j * 16, 16)]
            plsc.addupdate_scatter(hist_v, [idx16], ones16)
            return c2

        lax.fori_loop(0, per_w // 16, hbody, 0)
        pltpu.sync_copy(hist_v, cnt_hbm.at[pl.ds(wid * _N_PAD, _N_PAD)])

    return k(x, row, col)


def _sc_scatter(h0, h1, col, zeros_tile):
    mesh = plsc.VectorSubcoreMesh(core_axis_name="c", subcore_axis_name="s")

    @functools.partial(
        pl.kernel,
        out_type=jax.ShapeDtypeStruct((_NC, _N_HALF, _D), jnp.float32),
        mesh=mesh,
        scratch_types=[
            pltpu.VMEM((_CHA,), jnp.int32),
            pltpu.VMEM((_CHB,), jnp.int32),
            pltpu.VMEM((_CHA, _D), jnp.float32),
            pltpu.VMEM((_CHB, _D), jnp.float32),
            pltpu.VMEM_SHARED((_N_HALF + 8, _D), jnp.float32),
            pltpu.SemaphoreType.DMA,
            pltpu.SemaphoreType.DMA,
            pltpu.SemaphoreType.DMA,
            pltpu.SemaphoreType.DMA,
            pltpu.SemaphoreType.DMA,
            pltpu.SemaphoreType.DMA,
        ],
        compiler_params=pltpu.CompilerParams(needs_layout_passes=False),
    )
    def k(h0_hbm, h1_hbm, col_hbm, zero_hbm, out_hbm, idx_a, idx_b, rows_a,
          rows_b, acc_sh, sem_ia, sem_ib, sem_ra, sem_rb, sem_sa, sem_sb):
        c = lax.axis_index("c")
        s = lax.axis_index("s")
        nbase = c * _N_HALF
        bufs = ((idx_a, rows_a, _CHA, 0, sem_ia, sem_ra, sem_sa),
                (idx_b, rows_b, _CHB, _CHA, sem_ib, sem_rb, sem_sb))

        def wait_stream(bset):
            idx_v, rows_v = bset[0], bset[1]
            pltpu.make_async_copy(rows_v, acc_sh.at[idx_v], bset[6]).wait()

        def run_half(h_hbm, col_off, per_t):
            npair = per_t // (_CHA + _CHB)

            def start_dma(pair, bset):
                idx_v, rows_v, ch, off, sem_i, sem_r, _ = bset
                base = s * per_t + pair * (_CHA + _CHB) + off
                pltpu.async_copy(
                    col_hbm.at[pl.ds(col_off + base, ch)], idx_v, sem_i)
                pltpu.async_copy(h_hbm.at[pl.ds(base, ch)], rows_v, sem_r)

            def wait_dma(pair, bset):
                idx_v, rows_v, ch, off, sem_i, sem_r, _ = bset
                base = s * per_t + pair * (_CHA + _CHB) + off
                pltpu.make_async_copy(
                    col_hbm.at[pl.ds(col_off + base, ch)], idx_v,
                    sem_i).wait()
                pltpu.make_async_copy(
                    h_hbm.at[pl.ds(base, ch)], rows_v, sem_r).wait()

            start_dma(0, bufs[0])

            def body(io, carry):
                for b in range(2):
                    bset = bufs[b]
                    idx_v, rows_v, ch = bset[0], bset[1], bset[2]
                    sem_s = bset[6]
                    wait_dma(io, bset)

                    def tbody(j, c2):
                        c16 = idx_v[pl.ds(j * 16, 16)]
                        loc = c16 - nbase
                        ok = jnp.logical_and(loc >= 0, loc < _N_HALF)
                        idx_v[pl.ds(j * 16, 16)] = jnp.where(ok, loc, _N_HALF)
                        return c2

                    lax.fori_loop(0, ch // 16, tbody, 0)
                    pltpu.async_copy(rows_v, acc_sh.at[idx_v], sem_s,
                                     add=True)
                    other = bufs[1 - b]
                    if b == 0:
                        @pl.when(io >= 1)
                        def _():
                            wait_stream(other)

                        start_dma(io, other)
                    else:
                        @pl.when(io < npair - 1)
                        def _():
                            wait_stream(other)
                            start_dma(io + 1, other)

                return carry

            lax.fori_loop(0, npair, body, 0)
            wait_stream(bufs[0])
            wait_stream(bufs[1])

        pltpu.sync_copy(zero_hbm, acc_sh.at[pl.ds(s * _ROWS_T, _ROWS_T)])

        @pl.when(s == 0)
        def _():
            pltpu.sync_copy(zero_hbm.at[pl.ds(0, 8)],
                            acc_sh.at[pl.ds(_N_HALF, 8)])

        plsc.subcore_barrier()
        run_half(h0_hbm, 0, _EH0 // _NS)
        run_half(h1_hbm, _EH0, _EH1 // _NS)
        plsc.subcore_barrier()
        pltpu.sync_copy(
            acc_sh.at[pl.ds(s * _ROWS_T, _ROWS_T)],
            out_hbm.at[c].at[pl.ds(s * _ROWS_T, _ROWS_T)],
        )

    return k(h0, h1, col, zeros_tile)


def _edge_mlp(g, ea, w0x, w0e, b0, w1, b1, w2, b2, w3, b3, w4):
    eh = g.shape[0]
    BE = eh // 80
    grid = (eh // BE,)

    dot = functools.partial(jnp.dot, preferred_element_type=jnp.float32)
    bf = jnp.bfloat16

    def body(g_ref, e_ref, w0x_r, w0e_r, b0_r, w1_r, b1_r, w2_r, b2_r,
             w3_r, b3_r, w4_r, out_ref):
        g = g_ref[...].astype(bf)
        h = dot(g, w0x_r[...]) + dot(e_ref[...], w0e_r[...]) + b0_r[...]
        h = jnp.maximum(h, 0.0).astype(bf)
        h = jnp.maximum(dot(h, w1_r[...]) + b1_r[...], 0.0).astype(bf)
        h = jnp.maximum(dot(h, w2_r[...]) + b2_r[...], 0.0).astype(bf)
        h = jnp.maximum(dot(h, w3_r[...]) + b3_r[...], 0.0).astype(bf)
        out_ref[...] = dot(h, w4_r[...])

    full = lambda shape: pl.BlockSpec(shape, lambda i: (0,) * len(shape))
    return pl.pallas_call(
        body,
        grid=grid,
        in_specs=[
            pl.BlockSpec((BE, _D), lambda i: (i, 0)),
            pl.BlockSpec((BE, 16), lambda i: (i, 0)),
            full((_D, 128)), full((16, 128)), full((1, 128)),
            full((128, 128)), full((1, 128)),
            full((128, 128)), full((1, 128)),
            full((128, 144)), full((1, 144)),
            full((144, 128)),
        ],
        out_specs=pl.BlockSpec((BE, _D), lambda i: (i, 0)),
        out_shape=jax.ShapeDtypeStruct((eh, _D), jnp.float32),
    )(g, ea, w0x, w0e, b0, w1, b1, w2, b2, w3, b3, w4)


def _node_mlp(x, partials, cnt_parts, ga, w0x, w0g, b0, w1, b1, w2, b2,
              w3, b3):
    BN = 2048
    grid = (_N_PAD // BN,)

    def body(x_ref, p_ref, c_ref, ones_r, ga_r, w0x_r, w0g_r, b0_r, w1_r,
             b1_r, w2_r, b2_r, w3_r, b3_r, out_ref):
        s = p_ref[...]
        cnt = lax.dot_general(c_ref[...], ones_r[...],
                              (((0,), (0,)), ((), ())))
        agg = s * (1.0 / jnp.maximum(cnt, 1.0))
        bfc = jnp.bfloat16
        dot = functools.partial(jnp.dot, preferred_element_type=jnp.float32)
        gb = ga_r[...] @ w0g_r[...] + b0_r[...]
        h = dot(x_ref[...].astype(bfc), w0x_r[...].astype(bfc)) + agg + gb
        h = jnp.maximum(h, 0.0).astype(bfc)
        h = jnp.maximum(dot(h, w1_r[...].astype(bfc)) + b1_r[...],
                        0.0).astype(bfc)
        h = jnp.maximum(dot(h, w2_r[...].astype(bfc)) + b2_r[...],
                        0.0).astype(bfc)
        out_ref[...] = dot(h, w3_r[...].astype(bfc)) + b3_r[...]

    full = lambda shape: pl.BlockSpec(shape, lambda i: (0,) * len(shape))
    return pl.pallas_call(
        body,
        grid=grid,
        in_specs=[
            pl.BlockSpec((BN, _D), lambda i: (i, 0)),
            pl.BlockSpec((BN, _D), lambda i: (i, 0)),
            pl.BlockSpec((2 * _NW, BN), lambda i: (0, i)),
            full((2 * _NW, 1)),
            full((1, 16)),
            full((_D, 128)), full((16, 128)), full((1, 128)),
            full((128, 128)), full((1, 128)),
            full((128, 128)), full((1, 128)),
            full((128, 128)), full((1, 128)),
        ],
        out_specs=pl.BlockSpec((BN, 128), lambda i: (i, 0)),
        out_shape=jax.ShapeDtypeStruct((_N_PAD, 128), jnp.float32),
    )(x, partials, cnt_parts, jnp.ones((2 * _NW, 1), jnp.float32), ga, w0x,
      w0g, b0, w1, b1, w2, b2, w3, b3)


def kernel(x, edge_index, edge_attr, global_attr,
           W10, b10, W11, b11, W12, b12, W13, b13,
           W20, b20, W21, b21, W22, b22, W23, b23):
    row = edge_index[0]
    col = edge_index[1]

    w10x = W10[:_D]
    w10e = W10[_D:]
    w20x = W20[:_D]
    w20a = W20[_D:_D + 144]
    w20g = W20[_D + 144:]

    bf = jnp.bfloat16
    ea = edge_attr.astype(bf)
    ew = (w10x.astype(bf), w10e.astype(bf), b10.reshape(1, -1),
          W11.astype(bf), b11.reshape(1, -1),
          W12.astype(bf), b12.reshape(1, -1),
          W13.astype(bf), b13.reshape(1, -1),
          w20a.astype(bf))
    g0, cnt0 = _sc_gather_count(x, row, col, _EH0, 0)
    g1, cnt1 = _sc_gather_count(x, row, col, _EH1, _EH0)
    h0 = _edge_mlp(g0, ea[:_EH0], *ew)
    h1 = _edge_mlp(g1, ea[_EH0:], *ew)
    cnt_parts = jnp.concatenate([cnt0, cnt1]).reshape(2 * _NW, _N_PAD)
    zeros_tile = jnp.zeros((_ROWS_T, _D), jnp.float32)
    sums = _sc_scatter(h0, h1, col, zeros_tile).reshape(_N_PAD, _D)
    x_pad = jnp.concatenate(
        [x, jnp.zeros((_N_PAD - _N, _D), jnp.float32)], axis=0)
    out = _node_mlp(x_pad, sums, cnt_parts, global_attr,
                    w20x, w20g, b20.reshape(1, -1),
                    W21, b21.reshape(1, -1),
                    W22, b22.reshape(1, -1),
                    W23, b23.reshape(1, -1))
    return out[:_N]
